# initial kernel scaffold (unmeasured)
import functools

import jax
import jax.numpy as jnp
from jax import lax
from jax.experimental import pallas as pl
from jax.experimental.pallas import tpu as pltpu

N_DEV = 4


def kernel(x, w_mat):
    m_per, k = x.shape
    _, n_per = w_mat.shape

    def body(x_ref, w_ref, out_ref, comm_ref, send_sems, recv_sems):
        me = lax.axis_index("i")
        left = (me - 1) % N_DEV
        right = (me + 1) % N_DEV

        barrier_sem = pltpu.get_barrier_semaphore()
        for nbr in (left, right):
            pl.semaphore_signal(
                barrier_sem, inc=1,
                device_id=(nbr,), device_id_type=pl.DeviceIdType.MESH,
            )
        pl.semaphore_wait(barrier_sem, 2)

        def gemm(src, origin):
            acc = jnp.dot(src, w_ref[:, :], preferred_element_type=jnp.float32)
            out_ref[pl.ds(origin * m_per, m_per), :] = jnp.maximum(acc, 0.0)

        for h in range(N_DEV - 1):
            origin = (me - h) % N_DEV
            src = x_ref if h == 0 else comm_ref.at[origin]
            rdma = pltpu.make_async_remote_copy(
                src_ref=src,
                dst_ref=comm_ref.at[origin],
                send_sem=send_sems.at[h],
                recv_sem=recv_sems.at[h],
                device_id=(right,),
                device_id_type=pl.DeviceIdType.MESH,
            )
            rdma.start()
            if h == 0:
                gemm(x_ref[:, :], me)
            rdma.wait()
            recv_origin = (me - h - 1) % N_DEV
            gemm(comm_ref[recv_origin], recv_origin)

        @functools.partial(
            pl.run_scoped, second_barrier=pltpu.SemaphoreType.REGULAR
        )
        def _(second_barrier):
            for nbr in (left, right):
                pl.semaphore_signal(
                    second_barrier, inc=1,
                    device_id=(nbr,), device_id_type=pl.DeviceIdType.MESH,
                )
            pl.semaphore_wait(second_barrier, 2)

    return pl.pallas_call(
        body,
        out_shape=jax.ShapeDtypeStruct((N_DEV * m_per, n_per), jnp.float32),
        in_specs=[
            pl.BlockSpec(memory_space=pltpu.VMEM),
            pl.BlockSpec(memory_space=pltpu.VMEM),
        ],
        out_specs=pl.BlockSpec(memory_space=pltpu.VMEM),
        scratch_shapes=[
            pltpu.VMEM((N_DEV, m_per, k), x.dtype),
            pltpu.SemaphoreType.DMA((N_DEV - 1,)),
            pltpu.SemaphoreType.DMA((N_DEV - 1,)),
        ],
        compiler_params=pltpu.CompilerParams(collective_id=0),
    )(x, w_mat)


# baseline (device time: 306473 ns/iter reference)
import functools

import jax
import jax.numpy as jnp
from jax import lax
from jax.experimental import pallas as pl
from jax.experimental.pallas import tpu as pltpu

N_DEV = 4


def kernel(x, w_mat):
    m_per, k = x.shape
    _, n_per = w_mat.shape

    def body(x_ref, w_ref, out_ref, comm_ref, send_sems, recv_sems):
        me = lax.axis_index("i")
        left = (me - 1) % N_DEV
        right = (me + 1) % N_DEV

        barrier_sem = pltpu.get_barrier_semaphore()
        for nbr in (left, right):
            pl.semaphore_signal(
                barrier_sem, inc=1,
                device_id=(nbr,), device_id_type=pl.DeviceIdType.MESH,
            )
        pl.semaphore_wait(barrier_sem, 2)

        def gemm(src, origin):
            acc = jnp.dot(src, w_ref[:, :], preferred_element_type=jnp.float32)
            out_ref[pl.ds(origin * m_per, m_per), :] = jnp.maximum(acc, 0.0)

        for h in range(N_DEV - 1):
            src = x_ref if h == 0 else comm_ref.at[h - 1]
            rdma = pltpu.make_async_remote_copy(
                src_ref=src,
                dst_ref=comm_ref.at[h],
                send_sem=send_sems.at[h],
                recv_sem=recv_sems.at[h],
                device_id=(right,),
                device_id_type=pl.DeviceIdType.MESH,
            )
            rdma.start()
            if h == 0:
                gemm(x_ref[:, :], me)
            rdma.wait()
            gemm(comm_ref[h], (me - 1 - h) % N_DEV)

        @functools.partial(
            pl.run_scoped, second_barrier=pltpu.SemaphoreType.REGULAR
        )
        def _(second_barrier):
            for nbr in (left, right):
                pl.semaphore_signal(
                    second_barrier, inc=1,
                    device_id=(nbr,), device_id_type=pl.DeviceIdType.MESH,
                )
            pl.semaphore_wait(second_barrier, 2)

    return pl.pallas_call(
        body,
        out_shape=jax.ShapeDtypeStruct((N_DEV * m_per, n_per), jnp.float32),
        in_specs=[
            pl.BlockSpec(memory_space=pltpu.VMEM),
            pl.BlockSpec(memory_space=pltpu.VMEM),
        ],
        out_specs=pl.BlockSpec(memory_space=pltpu.VMEM),
        scratch_shapes=[
            pltpu.VMEM((N_DEV - 1, m_per, k), jnp.bfloat16),
            pltpu.SemaphoreType.DMA((N_DEV - 1,)),
            pltpu.SemaphoreType.DMA((N_DEV - 1,)),
        ],
        compiler_params=pltpu.CompilerParams(collective_id=0),
    )(x.astype(jnp.bfloat16), w_mat.astype(jnp.bfloat16))


# device time: 164463 ns/iter; 1.8635x vs baseline; 1.8635x over previous
import functools

import jax
import jax.numpy as jnp
from jax import lax
from jax.experimental import pallas as pl
from jax.experimental.pallas import tpu as pltpu

N_DEV = 4


def kernel(x, w_mat):
    m_per, k = x.shape
    _, n_per = w_mat.shape
    half_m = m_per // 2

    def body(x_ref, w_ref, out_ref, xb_ref, comm_ref, send_sems, recv_sems):
        me = lax.axis_index("i")
        left = (me - 1) % N_DEV
        right = (me + 1) % N_DEV
        opp = (me + 2) % N_DEV

        barrier_sem = pltpu.get_barrier_semaphore()
        for nbr in (left, right):
            pl.semaphore_signal(
                barrier_sem, inc=1,
                device_id=(nbr,), device_id_type=pl.DeviceIdType.MESH,
            )
        pl.semaphore_wait(barrier_sem, 2)

        def gemm(src, row_start):
            acc = jnp.dot(src, w_ref[:, :], preferred_element_type=jnp.float32)
            out_ref[pl.ds(row_start, src.shape[0]), :] = jnp.maximum(
                acc, 0.0
            ).astype(out_ref.dtype)

        def mk(src, dst, i, dev):
            return pltpu.make_async_remote_copy(
                src_ref=src, dst_ref=dst,
                send_sem=send_sems.at[i], recv_sem=recv_sems.at[i],
                device_id=(dev,), device_id_type=pl.DeviceIdType.MESH,
            )

        xb_ref[1, :, :] = x_ref[pl.ds(half_m, half_m), :].astype(jnp.bfloat16)
        r2 = mk(xb_ref.at[1], comm_ref.at[1, 1], 2, left)
        r2.start()
        xb_ref[0, :, :] = x_ref[pl.ds(0, half_m), :].astype(jnp.bfloat16)
        r0 = mk(xb_ref.at[0], comm_ref.at[0, 0], 0, right)
        r1 = mk(xb_ref.at[1], comm_ref.at[0, 1], 1, right)
        r3 = mk(xb_ref.at[0], comm_ref.at[1, 0], 3, left)
        r0.start()
        r1.start()
        r3.start()

        gemm(xb_ref[0], me * m_per)
        gemm(xb_ref[1], me * m_per + half_m)

        r0.wait_recv()
        r4 = mk(comm_ref.at[0, 0], comm_ref.at[2, 0], 4, right)
        r4.start()
        r2.wait_recv()
        r5 = mk(comm_ref.at[1, 1], comm_ref.at[2, 1], 5, left)
        r5.start()

        gemm(comm_ref[0, 0], left * m_per)
        gemm(comm_ref[1, 1], right * m_per + half_m)
        r1.wait_recv()
        gemm(comm_ref[0, 1], left * m_per + half_m)
        r3.wait_recv()
        gemm(comm_ref[1, 0], right * m_per)
        r4.wait_recv()
        gemm(comm_ref[2, 0], opp * m_per)
        r5.wait_recv()
        gemm(comm_ref[2, 1], opp * m_per + half_m)

        for r in (r0, r1, r2, r3, r4, r5):
            r.wait_send()

        @functools.partial(
            pl.run_scoped, second_barrier=pltpu.SemaphoreType.REGULAR
        )
        def _(second_barrier):
            for nbr in (left, right):
                pl.semaphore_signal(
                    second_barrier, inc=1,
                    device_id=(nbr,), device_id_type=pl.DeviceIdType.MESH,
                )
            pl.semaphore_wait(second_barrier, 2)

    return pl.pallas_call(
        body,
        out_shape=jax.ShapeDtypeStruct((N_DEV * m_per, n_per), jnp.bfloat16),
        in_specs=[
            pl.BlockSpec(memory_space=pltpu.VMEM),
            pl.BlockSpec(memory_space=pltpu.VMEM),
        ],
        out_specs=pl.BlockSpec(memory_space=pltpu.VMEM),
        scratch_shapes=[
            pltpu.VMEM((2, half_m, k), jnp.bfloat16),
            pltpu.VMEM((3, 2, half_m, k), jnp.bfloat16),
            pltpu.SemaphoreType.DMA((6,)),
            pltpu.SemaphoreType.DMA((6,)),
        ],
        compiler_params=pltpu.CompilerParams(
            collective_id=0, vmem_limit_bytes=60 * 1024 * 1024
        ),
    )(x, w_mat.astype(jnp.bfloat16))


# device time: 122662 ns/iter; 2.4985x vs baseline; 1.3408x over previous
import functools

import jax
import jax.numpy as jnp
from jax import lax
from jax.experimental import pallas as pl
from jax.experimental.pallas import tpu as pltpu

N_DEV = 4


def kernel(x, w_mat):
    m_per, k = x.shape
    _, n_per = w_mat.shape
    half_k = k // 2

    def body(x_ref, w_ref, out_ref, wg_ref, ob_ref, send_sems, recv_sems):
        me = lax.axis_index("i")
        left = (me - 1) % N_DEV
        right = (me + 1) % N_DEV
        opp = (me + 2) % N_DEV

        barrier_sem = pltpu.get_barrier_semaphore()
        for nbr in (left, right, opp):
            pl.semaphore_signal(
                barrier_sem, inc=1,
                device_id=(nbr,), device_id_type=pl.DeviceIdType.MESH,
            )
        pl.semaphore_wait(barrier_sem, 3)

        def mk(src, dst, i, dev):
            return pltpu.make_async_remote_copy(
                src_ref=src, dst_ref=dst,
                send_sem=send_sems.at[i], recv_sem=recv_sems.at[i],
                device_id=(dev,), device_id_type=pl.DeviceIdType.MESH,
            )

        def block(w0, w1):
            acc = jnp.dot(x_ref[:, pl.ds(0, half_k)], w0,
                          preferred_element_type=jnp.float32)
            acc += jnp.dot(x_ref[:, pl.ds(half_k, half_k)], w1,
                           preferred_element_type=jnp.float32)
            return jnp.maximum(acc, 0.0).astype(jnp.bfloat16)

        w_h0 = w_ref.at[pl.ds(0, half_k)]
        w_h1 = w_ref.at[pl.ds(half_k, half_k)]

        s0 = mk(w_h0, wg_ref.at[0, 0], 0, right)
        s1 = mk(w_h1, wg_ref.at[0, 1], 1, right)
        s2 = mk(w_h1, wg_ref.at[1, 1], 2, left)
        s3 = mk(w_h0, wg_ref.at[1, 0], 3, left)
        s0.start()
        s2.start()
        s1.start()
        s3.start()

        out_ref[pl.ds(me * m_per, m_per), :] = block(
            w_ref[pl.ds(0, half_k), :], w_ref[pl.ds(half_k, half_k), :]
        )

        s4 = None
        s0.wait_recv()
        s4 = mk(wg_ref.at[0, 0], wg_ref.at[2, 0], 4, right)
        s4.start()
        s2.wait_recv()
        s5 = mk(wg_ref.at[1, 1], wg_ref.at[2, 1], 5, left)
        s5.start()

        s3.wait_recv()
        ob_ref[0, :, :] = block(wg_ref[1, 0], wg_ref[1, 1])
        s6 = mk(ob_ref.at[0], out_ref.at[pl.ds(me * m_per, m_per)], 6, right)
        s6.start()

        s1.wait_recv()
        ob_ref[1, :, :] = block(wg_ref[0, 0], wg_ref[0, 1])
        s7 = mk(ob_ref.at[1], out_ref.at[pl.ds(me * m_per, m_per)], 7, left)
        s7.start()

        s4.wait_recv()
        s5.wait_recv()
        ob_ref[2, :, :] = block(wg_ref[2, 0], wg_ref[2, 1])
        s8 = mk(ob_ref.at[2], out_ref.at[pl.ds(me * m_per, m_per)], 8, opp)
        s8.start()

        s6.wait_recv()
        s7.wait_recv()
        s8.wait_recv()

        for s in (s0, s1, s2, s3, s4, s5, s6, s7, s8):
            s.wait_send()

        @functools.partial(
            pl.run_scoped, second_barrier=pltpu.SemaphoreType.REGULAR
        )
        def _(second_barrier):
            for nbr in (left, right, opp):
                pl.semaphore_signal(
                    second_barrier, inc=1,
                    device_id=(nbr,), device_id_type=pl.DeviceIdType.MESH,
                )
            pl.semaphore_wait(second_barrier, 3)

    return pl.pallas_call(
        body,
        out_shape=jax.ShapeDtypeStruct((N_DEV * m_per, n_per), jnp.bfloat16),
        in_specs=[
            pl.BlockSpec(memory_space=pltpu.VMEM),
            pl.BlockSpec(memory_space=pltpu.VMEM),
        ],
        out_specs=pl.BlockSpec(memory_space=pltpu.VMEM),
        scratch_shapes=[
            pltpu.VMEM((3, 2, half_k, n_per), jnp.bfloat16),
            pltpu.VMEM((3, m_per, n_per), jnp.bfloat16),
            pltpu.SemaphoreType.DMA((9,)),
            pltpu.SemaphoreType.DMA((9,)),
        ],
        compiler_params=pltpu.CompilerParams(
            collective_id=0, vmem_limit_bytes=60 * 1024 * 1024
        ),
    )(x.astype(jnp.bfloat16), w_mat.astype(jnp.bfloat16))


# device time: 115339 ns/iter; 2.6571x vs baseline; 1.0635x over previous
import functools

import jax
import jax.numpy as jnp
from jax import lax
from jax.experimental import pallas as pl
from jax.experimental.pallas import tpu as pltpu

N_DEV = 4


def kernel(x, w_mat):
    m_per, k = x.shape
    _, n_per = w_mat.shape
    half_k = k // 2

    def body(x_ref, w_ref, out_ref, xb_ref, wg_ref, ob_ref, send_sems,
             recv_sems):
        me = lax.axis_index("i")
        left = (me - 1) % N_DEV
        right = (me + 1) % N_DEV
        opp = (me + 2) % N_DEV

        barrier_sem = pltpu.get_barrier_semaphore()
        for nbr in (left, right, opp):
            pl.semaphore_signal(
                barrier_sem, inc=1,
                device_id=(nbr,), device_id_type=pl.DeviceIdType.MESH,
            )
        pl.semaphore_wait(barrier_sem, 3)

        def mk(src, dst, i, dev):
            return pltpu.make_async_remote_copy(
                src_ref=src, dst_ref=dst,
                send_sem=send_sems.at[i], recv_sem=recv_sems.at[i],
                device_id=(dev,), device_id_type=pl.DeviceIdType.MESH,
            )

        def block(w0, w1):
            acc = jnp.dot(xb_ref[:, pl.ds(0, half_k)], w0,
                          preferred_element_type=jnp.float32)
            acc += jnp.dot(xb_ref[:, pl.ds(half_k, half_k)], w1,
                           preferred_element_type=jnp.float32)
            return jnp.maximum(acc, 0.0).astype(jnp.bfloat16)

        w_h0 = w_ref.at[pl.ds(0, half_k)]
        w_h1 = w_ref.at[pl.ds(half_k, half_k)]

        s0 = mk(w_h0, wg_ref.at[0, 0], 0, right)
        s1 = mk(w_h1, wg_ref.at[0, 1], 1, right)
        s2 = mk(w_h1, wg_ref.at[1, 1], 2, left)
        s3 = mk(w_h0, wg_ref.at[1, 0], 3, left)
        s0.start()
        s2.start()
        s1.start()
        s3.start()

        xb_ref[:, :] = x_ref[:, :].astype(jnp.bfloat16)

        out_ref[pl.ds(me * m_per, m_per), :] = block(
            w_ref[pl.ds(0, half_k), :], w_ref[pl.ds(half_k, half_k), :]
        )

        s4 = None
        s0.wait_recv()
        s4 = mk(wg_ref.at[0, 0], wg_ref.at[2, 0], 4, right)
        s4.start()
        s2.wait_recv()
        s5 = mk(wg_ref.at[1, 1], wg_ref.at[2, 1], 5, left)
        s5.start()

        s3.wait_recv()
        ob_ref[0, :, :] = block(wg_ref[1, 0], wg_ref[1, 1])
        s6 = mk(ob_ref.at[0], out_ref.at[pl.ds(me * m_per, m_per)], 6, right)
        s6.start()

        s1.wait_recv()
        ob_ref[1, :, :] = block(wg_ref[0, 0], wg_ref[0, 1])
        s7 = mk(ob_ref.at[1], out_ref.at[pl.ds(me * m_per, m_per)], 7, left)
        s7.start()

        s4.wait_recv()
        s5.wait_recv()
        ob_ref[2, :, :] = block(wg_ref[2, 0], wg_ref[2, 1])
        s8 = mk(ob_ref.at[2], out_ref.at[pl.ds(me * m_per, m_per)], 8, opp)
        s8.start()

        s6.wait_recv()
        s7.wait_recv()
        s8.wait_recv()

        for s in (s0, s1, s2, s3, s4, s5, s6, s7, s8):
            s.wait_send()

        @functools.partial(
            pl.run_scoped, second_barrier=pltpu.SemaphoreType.REGULAR
        )
        def _(second_barrier):
            for nbr in (left, right, opp):
                pl.semaphore_signal(
                    second_barrier, inc=1,
                    device_id=(nbr,), device_id_type=pl.DeviceIdType.MESH,
                )
            pl.semaphore_wait(second_barrier, 3)

    return pl.pallas_call(
        body,
        out_shape=jax.ShapeDtypeStruct((N_DEV * m_per, n_per), jnp.bfloat16),
        in_specs=[
            pl.BlockSpec(memory_space=pltpu.VMEM),
            pl.BlockSpec(memory_space=pltpu.VMEM),
        ],
        out_specs=pl.BlockSpec(memory_space=pltpu.VMEM),
        scratch_shapes=[
            pltpu.VMEM((m_per, k), jnp.bfloat16),
            pltpu.VMEM((3, 2, half_k, n_per), jnp.bfloat16),
            pltpu.VMEM((3, m_per, n_per), jnp.bfloat16),
            pltpu.SemaphoreType.DMA((9,)),
            pltpu.SemaphoreType.DMA((9,)),
        ],
        compiler_params=pltpu.CompilerParams(
            collective_id=0, vmem_limit_bytes=60 * 1024 * 1024
        ),
    )(x, w_mat.astype(jnp.bfloat16))


# device time: 114529 ns/iter; 2.6759x vs baseline; 1.0071x over previous
import functools

import jax
import jax.numpy as jnp
from jax import lax
from jax.experimental import pallas as pl
from jax.experimental.pallas import tpu as pltpu

N_DEV = 4


def kernel(x, w_mat):
    m_per, k = x.shape
    _, n_per = w_mat.shape
    half_k = k // 2

    def body(x_ref, w_ref, out_ref, xb_ref, wg_ref, ob_ref, send_sems,
             recv_sems):
        me = lax.axis_index("i")
        left = (me - 1) % N_DEV
        right = (me + 1) % N_DEV
        opp = (me + 2) % N_DEV

        barrier_sem = pltpu.get_barrier_semaphore()
        for nbr in (left, right):
            pl.semaphore_signal(
                barrier_sem, inc=1,
                device_id=(nbr,), device_id_type=pl.DeviceIdType.MESH,
            )
        pl.semaphore_wait(barrier_sem, 2)

        def mk(src, dst, i, dev):
            return pltpu.make_async_remote_copy(
                src_ref=src, dst_ref=dst,
                send_sem=send_sems.at[i], recv_sem=recv_sems.at[i],
                device_id=(dev,), device_id_type=pl.DeviceIdType.MESH,
            )

        def block(w0, w1):
            acc = jnp.dot(xb_ref[:, pl.ds(0, half_k)], w0,
                          preferred_element_type=jnp.float32)
            acc += jnp.dot(xb_ref[:, pl.ds(half_k, half_k)], w1,
                           preferred_element_type=jnp.float32)
            return jnp.maximum(acc, 0.0).astype(jnp.bfloat16)

        w_h0 = w_ref.at[pl.ds(0, half_k)]
        w_h1 = w_ref.at[pl.ds(half_k, half_k)]

        s0 = mk(w_h0, wg_ref.at[0, 0], 0, right)
        s1 = mk(w_h1, wg_ref.at[0, 1], 1, right)
        s2 = mk(w_h1, wg_ref.at[1, 1], 2, left)
        s3 = mk(w_h0, wg_ref.at[1, 0], 3, left)
        s0.start()
        s2.start()
        s1.start()
        s3.start()

        xb_ref[:, :] = x_ref[:, :].astype(jnp.bfloat16)

        out_ref[pl.ds(me * m_per, m_per), :] = block(
            w_ref[pl.ds(0, half_k), :], w_ref[pl.ds(half_k, half_k), :]
        )

        s4 = None
        s0.wait_recv()
        s4 = mk(wg_ref.at[0, 0], wg_ref.at[2, 0], 4, right)
        s4.start()
        s2.wait_recv()
        s5 = mk(wg_ref.at[1, 1], wg_ref.at[2, 1], 5, left)
        s5.start()

        s3.wait_recv()
        ob_ref[0, :, :] = block(wg_ref[1, 0], wg_ref[1, 1])
        s6 = mk(ob_ref.at[0], out_ref.at[pl.ds(me * m_per, m_per)], 6, right)
        s6.start()

        s1.wait_recv()
        ob_ref[1, :, :] = block(wg_ref[0, 0], wg_ref[0, 1])
        s7 = mk(ob_ref.at[1], out_ref.at[pl.ds(me * m_per, m_per)], 7, left)
        s7.start()

        s4.wait_recv()
        s5.wait_recv()
        ob_ref[2, :, :] = block(wg_ref[2, 0], wg_ref[2, 1])
        s8 = mk(ob_ref.at[2], out_ref.at[pl.ds(me * m_per, m_per)], 8, opp)
        s8.start()

        s6.wait_recv()
        s7.wait_recv()
        s8.wait_recv()

        for s in (s0, s1, s2, s3, s4, s5, s6, s7, s8):
            s.wait_send()

        @functools.partial(
            pl.run_scoped, second_barrier=pltpu.SemaphoreType.REGULAR
        )
        def _(second_barrier):
            for nbr in (left, right):
                pl.semaphore_signal(
                    second_barrier, inc=1,
                    device_id=(nbr,), device_id_type=pl.DeviceIdType.MESH,
                )
            pl.semaphore_wait(second_barrier, 2)

    return pl.pallas_call(
        body,
        out_shape=jax.ShapeDtypeStruct((N_DEV * m_per, n_per), jnp.bfloat16),
        in_specs=[
            pl.BlockSpec(memory_space=pltpu.VMEM),
            pl.BlockSpec(memory_space=pltpu.VMEM),
        ],
        out_specs=pl.BlockSpec(memory_space=pltpu.VMEM),
        scratch_shapes=[
            pltpu.VMEM((m_per, k), jnp.bfloat16),
            pltpu.VMEM((3, 2, half_k, n_per), jnp.bfloat16),
            pltpu.VMEM((3, m_per, n_per), jnp.bfloat16),
            pltpu.SemaphoreType.DMA((9,)),
            pltpu.SemaphoreType.DMA((9,)),
        ],
        compiler_params=pltpu.CompilerParams(
            collective_id=0, vmem_limit_bytes=60 * 1024 * 1024
        ),
    )(x, w_mat.astype(jnp.bfloat16))


# device time: 86377 ns/iter; 3.5481x vs baseline; 1.3259x over previous
import functools

import jax
import jax.numpy as jnp
from jax import lax
from jax.experimental import pallas as pl
from jax.experimental.pallas import tpu as pltpu

N_DEV = 4


def kernel(x, w_mat):
    m_per, k = x.shape
    _, n_per = w_mat.shape
    half_k = k // 2

    def body(x_ref, w_ref, qw_ref, s_ref, out_ref, xb_ref, qwg_ref, sc_ref,
             ob_ref, send_sems, recv_sems):
        me = lax.axis_index("i")
        left = (me - 1) % N_DEV
        right = (me + 1) % N_DEV
        opp = (me + 2) % N_DEV

        barrier_sem = pltpu.get_barrier_semaphore()
        for nbr in (left, right, opp):
            pl.semaphore_signal(
                barrier_sem, inc=1,
                device_id=(nbr,), device_id_type=pl.DeviceIdType.MESH,
            )
        pl.semaphore_wait(barrier_sem, 3)

        def mk(src, dst, i, dev):
            return pltpu.make_async_remote_copy(
                src_ref=src, dst_ref=dst,
                send_sem=send_sems.at[i], recv_sem=recv_sems.at[i],
                device_id=(dev,), device_id_type=pl.DeviceIdType.MESH,
            )

        s9 = mk(s_ref, sc_ref.at[0], 9, right)
        s10 = mk(s_ref, sc_ref.at[1], 10, left)
        s11 = mk(s_ref, sc_ref.at[2], 11, opp)
        s9.start()
        s10.start()
        s11.start()
        qw_h0 = qw_ref.at[pl.ds(0, half_k)]
        qw_h1 = qw_ref.at[pl.ds(half_k, half_k)]
        s0 = mk(qw_h0, qwg_ref.at[0, 0], 0, right)
        s1 = mk(qw_h1, qwg_ref.at[0, 1], 1, right)
        s2 = mk(qw_h1, qwg_ref.at[1, 1], 2, left)
        s3 = mk(qw_h0, qwg_ref.at[1, 0], 3, left)
        s0.start()
        s2.start()
        s1.start()
        s3.start()

        xb_ref[:, :] = x_ref[:, :].astype(jnp.bfloat16)

        def dots(w0, w1):
            acc = jnp.dot(xb_ref[:, pl.ds(0, half_k)], w0,
                          preferred_element_type=jnp.float32)
            acc += jnp.dot(xb_ref[:, pl.ds(half_k, half_k)], w1,
                           preferred_element_type=jnp.float32)
            return acc

        def qblock(slot, sc_slot):
            acc = dots(qwg_ref[slot, 0].astype(jnp.bfloat16),
                       qwg_ref[slot, 1].astype(jnp.bfloat16))
            scale = sc_ref[sc_slot, 0, :] * (1.0 / 127.0)
            acc = acc * scale[None, :]
            return jnp.maximum(acc, 0.0).astype(jnp.bfloat16)

        acc = dots(w_ref[pl.ds(0, half_k), :], w_ref[pl.ds(half_k, half_k), :])
        out_ref[pl.ds(me * m_per, m_per), :] = jnp.maximum(
            acc, 0.0
        ).astype(jnp.bfloat16)

        s0.wait_recv()
        s4 = mk(qwg_ref.at[0, 0], qwg_ref.at[2, 0], 4, right)
        s4.start()
        s2.wait_recv()
        s5 = mk(qwg_ref.at[1, 1], qwg_ref.at[2, 1], 5, left)
        s5.start()

        s3.wait_recv()
        s10.wait_recv()
        ob_ref[0, :, :] = qblock(1, 1)
        s6 = mk(ob_ref.at[0], out_ref.at[pl.ds(me * m_per, m_per)], 6, right)
        s6.start()

        s1.wait_recv()
        s9.wait_recv()
        ob_ref[1, :, :] = qblock(0, 0)
        s7 = mk(ob_ref.at[1], out_ref.at[pl.ds(me * m_per, m_per)], 7, left)
        s7.start()

        s4.wait_recv()
        s5.wait_recv()
        s11.wait_recv()
        ob_ref[2, :, :] = qblock(2, 2)
        s8 = mk(ob_ref.at[2], out_ref.at[pl.ds(me * m_per, m_per)], 8, opp)
        s8.start()

        s6.wait_recv()
        s7.wait_recv()
        s8.wait_recv()

        for s in (s0, s1, s2, s3, s4, s5, s6, s7, s8, s9, s10, s11):
            s.wait_send()

        @functools.partial(
            pl.run_scoped, second_barrier=pltpu.SemaphoreType.REGULAR
        )
        def _(second_barrier):
            for nbr in (left, right):
                pl.semaphore_signal(
                    second_barrier, inc=1,
                    device_id=(nbr,), device_id_type=pl.DeviceIdType.MESH,
                )
            pl.semaphore_wait(second_barrier, 2)

    wb = w_mat.astype(jnp.bfloat16)
    s = jnp.maximum(jnp.max(jnp.abs(w_mat), axis=0), 1e-30)
    qw = jnp.round(w_mat * (127.0 / s)).astype(jnp.int8)
    s_tile = jnp.broadcast_to(s.astype(jnp.float32), (8, n_per))

    return pl.pallas_call(
        body,
        out_shape=jax.ShapeDtypeStruct((N_DEV * m_per, n_per), jnp.bfloat16),
        in_specs=[
            pl.BlockSpec(memory_space=pltpu.VMEM),
            pl.BlockSpec(memory_space=pltpu.VMEM),
            pl.BlockSpec(memory_space=pltpu.VMEM),
            pl.BlockSpec(memory_space=pltpu.VMEM),
        ],
        out_specs=pl.BlockSpec(memory_space=pltpu.VMEM),
        scratch_shapes=[
            pltpu.VMEM((m_per, k), jnp.bfloat16),
            pltpu.VMEM((3, 2, half_k, n_per), jnp.int8),
            pltpu.VMEM((3, 8, n_per), jnp.float32),
            pltpu.VMEM((3, m_per, n_per), jnp.bfloat16),
            pltpu.SemaphoreType.DMA((12,)),
            pltpu.SemaphoreType.DMA((12,)),
        ],
        compiler_params=pltpu.CompilerParams(
            collective_id=0, vmem_limit_bytes=60 * 1024 * 1024
        ),
    )(x, wb, qw, s_tile)
